# baseline (speedup 1.0000x reference)
import functools

import jax
import jax.numpy as jnp
from jax import lax
from jax.experimental import pallas as pl
from jax.experimental.pallas import tpu as pltpu
from jax.experimental.pallas import tpu_sc as plsc

N = 10000
E = 320000
D = 128
NC = 2
NS = 16
NW = NC * NS
EPW = E // NW
CH = 80
NCHUNK = EPW // CH
VCH = N // 10
CB = 200
NCB = VCH // CB

_mesh = plsc.VectorSubcoreMesh(core_axis_name="c", subcore_axis_name="s")


CH2 = 64
NCH2 = EPW // CH2
TAILE = EPW - NCH2 * CH2
DNB, DSK = 5, 2
MAIN2 = ((NCH2 - DSK) // DNB) * DNB


@functools.partial(
    pl.kernel,
    out_type=jax.ShapeDtypeStruct((NC * 2 * N,), jnp.float32),
    mesh=_mesh,
    scratch_types=[
        pltpu.VMEM((DNB, 2 * CH2), jnp.int32),
        pltpu.VMEM((2 * TAILE,), jnp.int32),
        pltpu.VMEM((2 * CH2,), jnp.float32),
        pltpu.VMEM((VCH,), jnp.float32),
        pltpu.VMEM_SHARED((2 * N,), jnp.float32),
        pltpu.SemaphoreType.DMA((DNB,)),
        pltpu.SemaphoreType.DMA,
    ],
)
def _deg_kernel(ei_hbm, srcn_hbm, zn_hbm, ones_hbm, out_hbm,
                ci_v, ct_v, ones_v, stage_v, acc_sh, isem, tsem):
    c = lax.axis_index("c")
    s = lax.axis_index("s")
    wid = s * NC + c
    base = wid * EPW
    pltpu.sync_copy(ones_hbm, ones_v)

    def loadidx(j, b):
        off = base + j * CH2
        pltpu.async_copy(ei_hbm.at[pl.ds(E + off, CH2)],
                         ci_v.at[b, pl.ds(0, CH2)], isem.at[b])
        pltpu.async_copy(srcn_hbm.at[pl.ds(off, CH2)],
                         ci_v.at[b, pl.ds(CH2, CH2)], isem.at[b])

    def wait_idx(b):
        pltpu.make_async_copy(ei_hbm.at[pl.ds(0, CH2)],
                              ci_v.at[b, pl.ds(0, CH2)], isem.at[b]).wait()
        pltpu.make_async_copy(ei_hbm.at[pl.ds(0, CH2)],
                              ci_v.at[b, pl.ds(CH2, CH2)], isem.at[b]).wait()

    def scat(b):
        pltpu.sync_copy(ones_v, acc_sh.at[ci_v.at[b]], add=True)

    for b in range(DSK):
        loadidx(b, b)

    @pl.when(s < 10)
    def _zero():
        pltpu.sync_copy(zn_hbm, stage_v)
        pltpu.sync_copy(stage_v, acc_sh.at[pl.ds(s * 2 * VCH, VCH)])
        pltpu.sync_copy(stage_v, acc_sh.at[pl.ds(s * 2 * VCH + VCH, VCH)])

    plsc.subcore_barrier()

    def group(g, carry):
        for b in range(DNB):
            j = g * DNB + b
            loadidx(j + DSK, (b + DSK) % DNB)
            wait_idx(b)
            scat(b)
        return carry

    lax.fori_loop(0, MAIN2 // DNB, group, 0)
    for j in range(MAIN2, NCH2):
        b = j % DNB
        if j + DSK < NCH2:
            loadidx(j + DSK, (b + DSK) % DNB)
        wait_idx(b)
        scat(b)

    toff = base + NCH2 * CH2
    pltpu.async_copy(ei_hbm.at[pl.ds(E + toff, TAILE)],
                     ct_v.at[pl.ds(0, TAILE)], tsem)
    pltpu.async_copy(srcn_hbm.at[pl.ds(toff, TAILE)],
                     ct_v.at[pl.ds(TAILE, TAILE)], tsem)
    pltpu.make_async_copy(ei_hbm.at[pl.ds(0, TAILE)],
                          ct_v.at[pl.ds(0, TAILE)], tsem).wait()
    pltpu.make_async_copy(ei_hbm.at[pl.ds(0, TAILE)],
                          ct_v.at[pl.ds(TAILE, TAILE)], tsem).wait()
    pltpu.sync_copy(ones_v.at[pl.ds(0, 2 * TAILE)], acc_sh.at[ct_v],
                    add=True)
    plsc.subcore_barrier()

    @pl.when(s < 10)
    def _writeback():
        pltpu.sync_copy(acc_sh.at[pl.ds(s * 2 * VCH, VCH)], stage_v)
        pltpu.sync_copy(stage_v,
                        out_hbm.at[pl.ds(c * 2 * N + s * 2 * VCH, VCH)])
        pltpu.sync_copy(acc_sh.at[pl.ds(s * 2 * VCH + VCH, VCH)], stage_v)
        pltpu.sync_copy(stage_v,
                        out_hbm.at[pl.ds(c * 2 * N + s * 2 * VCH + VCH,
                                         VCH)])


NBUF = 3
CHP = 80
NCHP = EPW // CHP
SKEW = 2
MAINC = ((NCHP - SKEW) // NBUF) * NBUF
WBCH = 80
WBF = 640
WBL = N - 15 * WBF
NZL = WBL // WBCH
NZF = WBF // WBCH


@functools.partial(
    pl.kernel,
    out_type=jax.ShapeDtypeStruct((NC, N, D), jnp.float32),
    mesh=_mesh,
    scratch_types=[
        pltpu.VMEM((EPW,), jnp.int32),
        pltpu.VMEM((NBUF, CHP), jnp.int32),
        pltpu.VMEM((NBUF, CHP, D), jnp.float32),
        pltpu.VMEM_SHARED((N, D), jnp.float32),
        pltpu.SemaphoreType.DMA,
        pltpu.SemaphoreType.DMA((NBUF,)),
        pltpu.SemaphoreType.DMA((NBUF,)),
        pltpu.SemaphoreType.DMA((2,)),
    ],
)
def _prop_kernel(y_hbm, ei_hbm, z_hbm, out_hbm,
                 si_v, di_v, rows_v, acc_sh, isem, dsem, gsem, wsem):
    c = lax.axis_index("c")
    s = lax.axis_index("s")
    wid = s * NC + c
    base = wid * EPW
    pltpu.async_copy(ei_hbm.at[pl.ds(base, EPW)], si_v, isem)
    wb0 = s * WBF

    pltpu.sync_copy(z_hbm, rows_v.at[0])
    for k in range(NZL):
        pltpu.sync_copy(rows_v.at[0],
                        acc_sh.at[pl.ds(wb0 + k * WBCH, WBCH)])

    @pl.when(s < 15)
    def _zero_rest():
        for k in range(NZL, NZF):
            pltpu.sync_copy(rows_v.at[0],
                            acc_sh.at[pl.ds(wb0 + k * WBCH, WBCH)])

    pltpu.make_async_copy(ei_hbm.at[pl.ds(base, EPW)], si_v, isem).wait()

    def gather(j, b):
        pltpu.async_copy(y_hbm.at[si_v.at[pl.ds(j * CHP, CHP)]],
                         rows_v.at[b], gsem.at[b])

    def load_didx(j, b):
        pltpu.async_copy(ei_hbm.at[pl.ds(E + base + j * CHP, CHP)],
                         di_v.at[b], dsem.at[b])

    def finish(j, b):
        pltpu.make_async_copy(y_hbm.at[pl.ds(0, CHP)],
                              rows_v.at[b], gsem.at[b]).wait()
        pltpu.make_async_copy(ei_hbm.at[pl.ds(0, CHP)], di_v.at[b],
                              dsem.at[b]).wait()
        pltpu.sync_copy(rows_v.at[b], acc_sh.at[di_v.at[b]], add=True)

    for b in range(SKEW):
        gather(b, b)
        load_didx(b, b)

    plsc.subcore_barrier()

    def group(g, carry):
        for b in range(NBUF):
            j = g * NBUF + b
            gather(j + SKEW, (b + SKEW) % NBUF)
            load_didx(j + SKEW, (b + SKEW) % NBUF)
            finish(j, b)
        return carry

    lax.fori_loop(0, MAINC // NBUF, group, 0)
    for j in range(MAINC, NCHP):
        b = j % NBUF
        if j + SKEW < NCHP:
            gather(j + SKEW, (b + SKEW) % NBUF)
            load_didx(j + SKEW, (b + SKEW) % NBUF)
        finish(j, b)
    plsc.subcore_barrier()

    def wb_one(k):
        t = k % 2
        if k >= 2:
            pltpu.make_async_copy(
                rows_v.at[t],
                out_hbm.at[c, pl.ds(wb0 + (k - 2) * WBCH, WBCH)],
                wsem.at[t]).wait()
        pltpu.sync_copy(acc_sh.at[pl.ds(wb0 + k * WBCH, WBCH)],
                        rows_v.at[t])
        pltpu.async_copy(rows_v.at[t],
                         out_hbm.at[c, pl.ds(wb0 + k * WBCH, WBCH)],
                         wsem.at[t])

    def wb_drain(nk):
        for k in (nk - 2, nk - 1):
            pltpu.make_async_copy(
                rows_v.at[k % 2],
                out_hbm.at[c, pl.ds(wb0 + k * WBCH, WBCH)],
                wsem.at[k % 2]).wait()

    @pl.when(s < 15)
    def _writeback_f():
        for k in range(NZF):
            wb_one(k)
        wb_drain(NZF)

    @pl.when(s == 15)
    def _writeback_l():
        for k in range(NZL):
            wb_one(k)
        wb_drain(NZL)


BN = 2000
G = N // BN

_row = pl.BlockSpec((BN, D), lambda i: (i, 0))
_part = pl.BlockSpec((NC, BN, D), lambda i: (0, i, 0))
_col = pl.BlockSpec((BN, 1), lambda i: (i, 0))
_wfull = pl.BlockSpec((D, D), lambda i: (0, 0))
_wc3 = pl.BlockSpec((3, D, D), lambda i: (0, 0, 0))
_bias = pl.BlockSpec((1, D), lambda i: (0, 0))

_rowD = jax.ShapeDtypeStruct((N, D), jnp.float32)


def _k1_body(h_ref, w_ref, os_ref, o_ref):
    o_ref[...] = jnp.dot(h_ref[...], w_ref[...],
                         preferred_element_type=jnp.float32) * os_ref[...]


_k1 = pl.pallas_call(
    _k1_body, grid=(G,),
    in_specs=[_row, _wfull, _col], out_specs=_row, out_shape=_rowD)


def _k2_body(sp_ref, is_ref, b_ref, o_ref):
    ssum = sp_ref[0] + sp_ref[1]
    o_ref[...] = jnp.maximum(ssum * is_ref[...] + b_ref[...], 0.0)


_k2 = pl.pallas_call(
    _k2_body, grid=(G,),
    in_specs=[_part, _col, _bias], out_specs=_row, out_shape=_rowD)


def _k3_body(x1_ref, sp_ref, ws_ref, wn_ref, ii_ref, is_ref, b_ref,
             x2_ref, z_ref):
    neigh = (sp_ref[0] + sp_ref[1]) * ii_ref[...]
    x2 = (jnp.dot(x1_ref[...], ws_ref[...], preferred_element_type=jnp.float32)
          + jnp.dot(neigh, wn_ref[...], preferred_element_type=jnp.float32)
          + b_ref[...])
    x2 = jnp.maximum(x2, 0.0)
    x2_ref[...] = x2
    z_ref[...] = x2 * is_ref[...]


_k3 = pl.pallas_call(
    _k3_body, grid=(G,),
    in_specs=[_row, _part, _wfull, _wfull, _col, _col, _bias],
    out_specs=[_row, _row], out_shape=[_rowD, _rowD])


def _k4_body(sp_ref, x2_ref, wc_ref, is_ref, oacc_ref, w_ref):
    t1 = -(sp_ref[0] + sp_ref[1]) * is_ref[...]
    oacc_ref[...] = (
        jnp.dot(x2_ref[...], wc_ref[0], preferred_element_type=jnp.float32)
        + jnp.dot(t1, wc_ref[1], preferred_element_type=jnp.float32))
    w_ref[...] = t1 * is_ref[...]


_k4 = pl.pallas_call(
    _k4_body, grid=(G,),
    in_specs=[_part, _row, _wc3, _col],
    out_specs=[_row, _row], out_shape=[_rowD, _rowD])


def _k5_body(oacc_ref, sp_ref, x2_ref, wc_ref, is_ref, bc_ref, o_ref):
    t2 = -2.0 * (sp_ref[0] + sp_ref[1]) * is_ref[...] - x2_ref[...]
    o_ref[...] = (oacc_ref[...]
                  + jnp.dot(t2, wc_ref[2], preferred_element_type=jnp.float32)
                  + bc_ref[...])


_k5 = pl.pallas_call(
    _k5_body, grid=(G,),
    in_specs=[_row, _part, _row, _wc3, _col, _bias],
    out_specs=_row, out_shape=_rowD)


def kernel(h, edge_index, W1, b1, Ws, Wn, b2, Wc, bc):
    ei = edge_index.astype(jnp.int32).reshape(2 * E)
    srcn = ei[:E] + N
    zN = jnp.zeros((VCH,), jnp.float32)
    zND = jnp.zeros((WBCH, D), jnp.float32)
    onesC = jnp.ones((2 * CH2,), jnp.float32)

    deg = _deg_kernel(ei, srcn, zN, onesC).reshape(NC, 2, N)
    in_cnt = jnp.maximum(deg[0, 0] + deg[1, 0], 1.0)
    out_cnt = jnp.maximum(deg[0, 1] + deg[1, 1], 1.0)
    in_is = lax.rsqrt(in_cnt)[:, None]
    inv_in = (1.0 / in_cnt)[:, None]
    out_is = lax.rsqrt(out_cnt)[:, None]

    y1 = _k1(h, W1, out_is)
    s1 = _prop_kernel(y1, ei, zND)
    x1 = _k2(s1, in_is, b1[None, :])
    s2 = _prop_kernel(x1, ei, zND)
    x2, z = _k3(x1, s2, Ws, Wn, inv_in, in_is, b2[None, :])
    s3 = _prop_kernel(z, ei, zND)
    oacc, w = _k4(s3, x2, Wc, in_is)
    s4 = _prop_kernel(w, ei, zND)
    return _k5(oacc, s4, x2, Wc, in_is, bc[None, :])


# speedup vs baseline: 1.0446x; 1.0446x over previous
import functools

import jax
import jax.numpy as jnp
from jax import lax
from jax.experimental import pallas as pl
from jax.experimental.pallas import tpu as pltpu
from jax.experimental.pallas import tpu_sc as plsc

N = 10000
E = 320000
D = 128
NC = 2
NS = 16
NW = NC * NS
EPW = E // NW
CH = 80
NCHUNK = EPW // CH
VCH = N // 10
CB = 200
NCB = VCH // CB

_mesh = plsc.VectorSubcoreMesh(core_axis_name="c", subcore_axis_name="s")


@functools.partial(
    pl.kernel,
    out_type=jax.ShapeDtypeStruct((NC * 2 * N,), jnp.float32),
    mesh=_mesh,
    scratch_types=[
        pltpu.VMEM((5, CH), jnp.int32),
        pltpu.VMEM((5, CH), jnp.int32),
        pltpu.VMEM((CH,), jnp.float32),
        pltpu.VMEM((VCH,), jnp.float32),
        pltpu.VMEM_SHARED((N,), jnp.float32),
        pltpu.VMEM_SHARED((N,), jnp.float32),
        pltpu.SemaphoreType.DMA((5,)),
        pltpu.SemaphoreType.DMA((5,)),
    ],
)
def _deg_kernel(ei_hbm, zn_hbm, ones_hbm, out_hbm,
                si_v, di_v, ones_v, stage_v, in_sh, out_sh, isem, ssem):
    c = lax.axis_index("c")
    s = lax.axis_index("s")
    wid = s * NC + c
    base = wid * EPW
    DNB, DSK = 5, 2
    pltpu.sync_copy(ones_hbm, ones_v)

    def loadidx(j, b):
        off = base + j * CH
        pltpu.async_copy(ei_hbm.at[pl.ds(off, CH)], si_v.at[b],
                         isem.at[b])
        pltpu.async_copy(ei_hbm.at[pl.ds(E + off, CH)], di_v.at[b],
                         isem.at[b])

    def wait_idx(b):
        pltpu.make_async_copy(ei_hbm.at[pl.ds(0, CH)], si_v.at[b],
                              isem.at[b]).wait()
        pltpu.make_async_copy(ei_hbm.at[pl.ds(0, CH)], di_v.at[b],
                              isem.at[b]).wait()

    def scat(b):
        pltpu.sync_copy(ones_v, in_sh.at[di_v.at[b]], add=True)
        pltpu.sync_copy(ones_v, out_sh.at[si_v.at[b]], add=True)

    for b in range(DSK):
        loadidx(b, b)

    @pl.when(s < 10)
    def _zero():
        pltpu.sync_copy(zn_hbm, stage_v)
        pltpu.sync_copy(stage_v, in_sh.at[pl.ds(s * VCH, VCH)])
        pltpu.sync_copy(stage_v, out_sh.at[pl.ds(s * VCH, VCH)])

    plsc.subcore_barrier()

    def group(g, carry):
        for b in range(DNB):
            j = g * DNB + b
            loadidx(j + DSK, (b + DSK) % DNB)
            wait_idx(b)
            scat(b)
        return carry

    lax.fori_loop(0, NCHUNK // DNB - 1, group, 0)
    for j in range(NCHUNK - DNB, NCHUNK):
        b = j % DNB
        if j + DSK < NCHUNK:
            loadidx(j + DSK, (b + DSK) % DNB)
        wait_idx(b)
        scat(b)
    plsc.subcore_barrier()

    @pl.when(s < 10)
    def _writeback():
        pltpu.sync_copy(in_sh.at[pl.ds(s * VCH, VCH)], stage_v)
        pltpu.sync_copy(stage_v, out_hbm.at[pl.ds(c * 2 * N + s * VCH, VCH)])
        pltpu.sync_copy(out_sh.at[pl.ds(s * VCH, VCH)], stage_v)
        pltpu.sync_copy(stage_v,
                        out_hbm.at[pl.ds(c * 2 * N + N + s * VCH, VCH)])


NBUF = 3
CHP = 80
NCHP = EPW // CHP
SKEW = 2
MAINC = ((NCHP - SKEW) // NBUF) * NBUF
WBCH = 80
WBF = 640
WBL = N - 15 * WBF
NZL = WBL // WBCH
NZF = WBF // WBCH


@functools.partial(
    pl.kernel,
    out_type=jax.ShapeDtypeStruct((NC, N, D), jnp.float32),
    mesh=_mesh,
    scratch_types=[
        pltpu.VMEM((EPW,), jnp.int32),
        pltpu.VMEM((NBUF, CHP), jnp.int32),
        pltpu.VMEM((NBUF, CHP, D), jnp.float32),
        pltpu.VMEM_SHARED((N, D), jnp.float32),
        pltpu.SemaphoreType.DMA,
        pltpu.SemaphoreType.DMA((NBUF,)),
        pltpu.SemaphoreType.DMA((NBUF,)),
        pltpu.SemaphoreType.DMA((2,)),
    ],
)
def _prop_kernel(y_hbm, ei_hbm, z_hbm, out_hbm,
                 si_v, di_v, rows_v, acc_sh, isem, dsem, gsem, wsem):
    c = lax.axis_index("c")
    s = lax.axis_index("s")
    wid = s * NC + c
    base = wid * EPW
    pltpu.async_copy(ei_hbm.at[pl.ds(base, EPW)], si_v, isem)
    wb0 = s * WBF

    pltpu.sync_copy(z_hbm, rows_v.at[0])
    for k in range(NZL):
        pltpu.sync_copy(rows_v.at[0],
                        acc_sh.at[pl.ds(wb0 + k * WBCH, WBCH)])

    @pl.when(s < 15)
    def _zero_rest():
        for k in range(NZL, NZF):
            pltpu.sync_copy(rows_v.at[0],
                            acc_sh.at[pl.ds(wb0 + k * WBCH, WBCH)])

    pltpu.make_async_copy(ei_hbm.at[pl.ds(base, EPW)], si_v, isem).wait()

    def gather(j, b):
        pltpu.async_copy(y_hbm.at[si_v.at[pl.ds(j * CHP, CHP)]],
                         rows_v.at[b], gsem.at[b])

    def load_didx(j, b):
        pltpu.async_copy(ei_hbm.at[pl.ds(E + base + j * CHP, CHP)],
                         di_v.at[b], dsem.at[b])

    def finish(j, b):
        pltpu.make_async_copy(y_hbm.at[pl.ds(0, CHP)],
                              rows_v.at[b], gsem.at[b]).wait()
        pltpu.make_async_copy(ei_hbm.at[pl.ds(0, CHP)], di_v.at[b],
                              dsem.at[b]).wait()
        pltpu.sync_copy(rows_v.at[b], acc_sh.at[di_v.at[b]], add=True)

    for b in range(SKEW):
        gather(b, b)
        load_didx(b, b)

    plsc.subcore_barrier()

    def group(g, carry):
        for b in range(NBUF):
            j = g * NBUF + b
            gather(j + SKEW, (b + SKEW) % NBUF)
            load_didx(j + SKEW, (b + SKEW) % NBUF)
            finish(j, b)
        return carry

    lax.fori_loop(0, MAINC // NBUF, group, 0)
    for j in range(MAINC, NCHP):
        b = j % NBUF
        if j + SKEW < NCHP:
            gather(j + SKEW, (b + SKEW) % NBUF)
            load_didx(j + SKEW, (b + SKEW) % NBUF)
        finish(j, b)
    plsc.subcore_barrier()

    def wb_one(k):
        t = k % 2
        if k >= 2:
            pltpu.make_async_copy(
                rows_v.at[t],
                out_hbm.at[c, pl.ds(wb0 + (k - 2) * WBCH, WBCH)],
                wsem.at[t]).wait()
        pltpu.sync_copy(acc_sh.at[pl.ds(wb0 + k * WBCH, WBCH)],
                        rows_v.at[t])
        pltpu.async_copy(rows_v.at[t],
                         out_hbm.at[c, pl.ds(wb0 + k * WBCH, WBCH)],
                         wsem.at[t])

    def wb_drain(nk):
        for k in (nk - 2, nk - 1):
            pltpu.make_async_copy(
                rows_v.at[k % 2],
                out_hbm.at[c, pl.ds(wb0 + k * WBCH, WBCH)],
                wsem.at[k % 2]).wait()

    @pl.when(s < 15)
    def _writeback_f():
        for k in range(NZF):
            wb_one(k)
        wb_drain(NZF)

    @pl.when(s == 15)
    def _writeback_l():
        for k in range(NZL):
            wb_one(k)
        wb_drain(NZL)


BN = 5000
G = N // BN

_row = pl.BlockSpec((BN, D), lambda i: (i, 0))
_part = pl.BlockSpec((NC, BN, D), lambda i: (0, i, 0))
_col = pl.BlockSpec((BN, 1), lambda i: (i, 0))
_wfull = pl.BlockSpec((D, D), lambda i: (0, 0))
_wc3 = pl.BlockSpec((3, D, D), lambda i: (0, 0, 0))
_bias = pl.BlockSpec((1, D), lambda i: (0, 0))

_rowD = jax.ShapeDtypeStruct((N, D), jnp.float32)


def _k1_body(h_ref, w_ref, os_ref, o_ref):
    o_ref[...] = jnp.dot(h_ref[...], w_ref[...],
                         preferred_element_type=jnp.float32) * os_ref[...]


_k1 = pl.pallas_call(
    _k1_body, grid=(G,),
    in_specs=[_row, _wfull, _col], out_specs=_row, out_shape=_rowD)


def _k2_body(sp_ref, is_ref, b_ref, o_ref):
    ssum = sp_ref[0] + sp_ref[1]
    o_ref[...] = jnp.maximum(ssum * is_ref[...] + b_ref[...], 0.0)


_k2 = pl.pallas_call(
    _k2_body, grid=(G,),
    in_specs=[_part, _col, _bias], out_specs=_row, out_shape=_rowD)


def _k3_body(x1_ref, sp_ref, ws_ref, wn_ref, ii_ref, is_ref, b_ref,
             x2_ref, z_ref):
    neigh = (sp_ref[0] + sp_ref[1]) * ii_ref[...]
    x2 = (jnp.dot(x1_ref[...], ws_ref[...], preferred_element_type=jnp.float32)
          + jnp.dot(neigh, wn_ref[...], preferred_element_type=jnp.float32)
          + b_ref[...])
    x2 = jnp.maximum(x2, 0.0)
    x2_ref[...] = x2
    z_ref[...] = x2 * is_ref[...]


_k3 = pl.pallas_call(
    _k3_body, grid=(G,),
    in_specs=[_row, _part, _wfull, _wfull, _col, _col, _bias],
    out_specs=[_row, _row], out_shape=[_rowD, _rowD])


def _k4_body(sp_ref, x2_ref, wc_ref, is_ref, oacc_ref, w_ref):
    t1 = -(sp_ref[0] + sp_ref[1]) * is_ref[...]
    oacc_ref[...] = (
        jnp.dot(x2_ref[...], wc_ref[0], preferred_element_type=jnp.float32)
        + jnp.dot(t1, wc_ref[1], preferred_element_type=jnp.float32))
    w_ref[...] = t1 * is_ref[...]


_k4 = pl.pallas_call(
    _k4_body, grid=(G,),
    in_specs=[_part, _row, _wc3, _col],
    out_specs=[_row, _row], out_shape=[_rowD, _rowD])


def _k5_body(oacc_ref, sp_ref, x2_ref, wc_ref, is_ref, bc_ref, o_ref):
    t2 = -2.0 * (sp_ref[0] + sp_ref[1]) * is_ref[...] - x2_ref[...]
    o_ref[...] = (oacc_ref[...]
                  + jnp.dot(t2, wc_ref[2], preferred_element_type=jnp.float32)
                  + bc_ref[...])


_k5 = pl.pallas_call(
    _k5_body, grid=(G,),
    in_specs=[_row, _part, _row, _wc3, _col, _bias],
    out_specs=_row, out_shape=_rowD)


def kernel(h, edge_index, W1, b1, Ws, Wn, b2, Wc, bc):
    ei = edge_index.astype(jnp.int32).reshape(2 * E)
    zN = jnp.zeros((VCH,), jnp.float32)
    zND = jnp.zeros((WBCH, D), jnp.float32)
    onesC = jnp.ones((CH,), jnp.float32)

    deg = _deg_kernel(ei, zN, onesC).reshape(NC, 2, N)
    in_cnt = jnp.maximum(deg[0, 0] + deg[1, 0], 1.0)
    out_cnt = jnp.maximum(deg[0, 1] + deg[1, 1], 1.0)
    in_is = lax.rsqrt(in_cnt)[:, None]
    inv_in = (1.0 / in_cnt)[:, None]
    out_is = lax.rsqrt(out_cnt)[:, None]

    y1 = _k1(h, W1, out_is)
    s1 = _prop_kernel(y1, ei, zND)
    x1 = _k2(s1, in_is, b1[None, :])
    s2 = _prop_kernel(x1, ei, zND)
    x2, z = _k3(x1, s2, Ws, Wn, inv_in, in_is, b2[None, :])
    s3 = _prop_kernel(z, ei, zND)
    oacc, w = _k4(s3, x2, Wc, in_is)
    s4 = _prop_kernel(w, ei, zND)
    return _k5(oacc, s4, x2, Wc, in_is, bc[None, :])
